# baseline (speedup 1.0000x reference)
import functools

import jax
import jax.numpy as jnp
from jax import lax
from jax.experimental import pallas as pl
from jax.experimental.pallas import tpu as pltpu
from jax.experimental.pallas import tpu_sc as plsc

N = 5000
NPAD = 5120
TIB = 512
TJ = 512
NJT = NPAD // TJ
CONF_BAR = 0.1
NMS_RADIUS = 0.01
TOPK = 512
NW = 32
CHUNK = NPAD // NW
L = 16

_MESH = plsc.VectorSubcoreMesh(core_axis_name="c", subcore_axis_name="s")


def _wid():
    return lax.axis_index("s") * 2 + lax.axis_index("c")


def _sc_gather_body(nn1_hbm, nn2_hbm, a2_hbm, b2_hbm, sc_hbm,
                    smc_out, a2g_out, b2g_out,
                    nn2_v, a2_v, b2_v, nn1_v, sc_v, smc_v, a2g_v, b2g_v):
    wid = _wid()
    base = wid * CHUNK
    pltpu.sync_copy(nn2_hbm, nn2_v)
    pltpu.sync_copy(a2_hbm, a2_v)
    pltpu.sync_copy(b2_hbm, b2_v)
    pltpu.sync_copy(nn1_hbm.at[pl.ds(base, CHUNK)], nn1_v)
    pltpu.sync_copy(sc_hbm.at[pl.ds(base, CHUNK)], sc_v)

    def chunk(k, carry):
        sl = pl.ds(k * L, L)
        idx = nn1_v[sl]
        back = plsc.load_gather(nn2_v, [idx])
        iv = base + k * L + lax.iota(jnp.int32, L)
        smc_v[sl] = jnp.where(back != iv, -1.0, sc_v[sl])
        a2g_v[sl] = plsc.load_gather(a2_v, [idx])
        b2g_v[sl] = plsc.load_gather(b2_v, [idx])
        return carry

    lax.fori_loop(0, CHUNK // L, chunk, 0)
    pltpu.sync_copy(smc_v, smc_out.at[pl.ds(base, CHUNK)])
    pltpu.sync_copy(a2g_v, a2g_out.at[pl.ds(base, CHUNK)])
    pltpu.sync_copy(b2g_v, b2g_out.at[pl.ds(base, CHUNK)])


_sc_gather = functools.partial(
    pl.kernel, _sc_gather_body,
    out_type=[jax.ShapeDtypeStruct((NPAD,), jnp.float32)] * 3,
    mesh=_MESH,
    compiler_params=pltpu.CompilerParams(needs_layout_passes=False),
    scratch_types=[
        pltpu.VMEM((NPAD,), jnp.int32),
        pltpu.VMEM((NPAD,), jnp.float32),
        pltpu.VMEM((NPAD,), jnp.float32),
        pltpu.VMEM((CHUNK,), jnp.int32),
        pltpu.VMEM((CHUNK,), jnp.float32),
        pltpu.VMEM((CHUNK,), jnp.float32),
        pltpu.VMEM((CHUNK,), jnp.float32),
        pltpu.VMEM((CHUNK,), jnp.float32),
    ],
)()


def _sc_compact_body(smc_hbm, a1_hbm, b1_hbm, a2g_hbm, b2g_hbm,
                     ja_out, sa_out, a1a_out, b1a_out, a2a_out, b2a_out,
                     kc_out,
                     s_v, a1_v, b1_v, a2_v, b2_v,
                     ja_v, sa_v, a1a_v, b1a_v, a2a_v, b2a_v, kc_v):
    @pl.when(_wid() == 0)
    def _():
        pltpu.sync_copy(smc_hbm, s_v)
        pltpu.sync_copy(a1_hbm, a1_v)
        pltpu.sync_copy(b1_hbm, b1_v)
        pltpu.sync_copy(a2g_hbm, a2_v)
        pltpu.sync_copy(b2g_hbm, b2_v)

        zi = jnp.zeros((L,), jnp.int32)
        zf = jnp.zeros((L,), jnp.float32)

        def fill(k, carry):
            sl = pl.ds(k * L, L)
            ja_v[sl] = zi
            sa_v[sl] = zf - 3.0
            a1a_v[sl] = zf
            return carry

        lax.fori_loop(0, NPAD // L, fill, 0)

        def compact(k, c):
            sl = pl.ds(k * L, L)
            sv = s_v[sl]
            m = sv >= CONF_BAR
            mi = m.astype(jnp.int32)
            incl = plsc.cumsum(mi)
            slot = c + incl - mi
            iv = k * L + lax.iota(jnp.int32, L)
            plsc.store_scatter(ja_v, [slot], iv, mask=m)
            plsc.store_scatter(sa_v, [slot], sv, mask=m)
            plsc.store_scatter(a1a_v, [slot], a1_v[sl], mask=m)
            plsc.store_scatter(b1a_v, [slot], b1_v[sl], mask=m)
            plsc.store_scatter(a2a_v, [slot], a2_v[sl], mask=m)
            plsc.store_scatter(b2a_v, [slot], b2_v[sl], mask=m)
            return c + jnp.sum(mi)

        kcount = lax.fori_loop(0, NPAD // L, compact, jnp.int32(0))
        kc_v[...] = jnp.zeros((L,), jnp.int32) + kcount
        pltpu.sync_copy(ja_v, ja_out)
        pltpu.sync_copy(sa_v, sa_out)
        pltpu.sync_copy(a1a_v, a1a_out)
        pltpu.sync_copy(b1a_v, b1a_out)
        pltpu.sync_copy(a2a_v, a2a_out)
        pltpu.sync_copy(b2a_v, b2a_out)
        pltpu.sync_copy(kc_v, kc_out)


_sc_compact = functools.partial(
    pl.kernel, _sc_compact_body,
    out_type=[jax.ShapeDtypeStruct((NPAD,), jnp.int32)]
    + [jax.ShapeDtypeStruct((NPAD,), jnp.float32)] * 5
    + [jax.ShapeDtypeStruct((L,), jnp.int32)],
    mesh=_MESH,
    compiler_params=pltpu.CompilerParams(needs_layout_passes=False),
    scratch_types=[
        pltpu.VMEM((NPAD,), jnp.float32),
        pltpu.VMEM((NPAD,), jnp.float32),
        pltpu.VMEM((NPAD,), jnp.float32),
        pltpu.VMEM((NPAD,), jnp.float32),
        pltpu.VMEM((NPAD,), jnp.float32),
        pltpu.VMEM((NPAD,), jnp.int32),
        pltpu.VMEM((NPAD,), jnp.float32),
        pltpu.VMEM((NPAD,), jnp.float32),
        pltpu.VMEM((NPAD,), jnp.float32),
        pltpu.VMEM((NPAD,), jnp.float32),
        pltpu.VMEM((NPAD,), jnp.float32),
        pltpu.VMEM((L,), jnp.int32),
    ],
)()


def _sc_fgather_body(f_hbm, ja_hbm, kc_hbm, fa_out,
                     f_v, ja_v, kc_v, fa_v):
    wid = _wid()
    base = wid * CHUNK
    pltpu.sync_copy(f_hbm, f_v)
    pltpu.sync_copy(ja_hbm.at[pl.ds(base, CHUNK)], ja_v)
    pltpu.sync_copy(kc_hbm, kc_v)
    kvec = kc_v[...]

    def chunk(k, carry):
        sl = pl.ds(k * L, L)
        jv = ja_v[sl]
        kg = base + k * L + lax.iota(jnp.int32, L)
        m = kg < kvec
        vals = plsc.load_gather(f_v, [jnp.where(m, jv, 0)])
        fa_v[sl] = jnp.where(m, vals, -3.0)
        return carry

    lax.fori_loop(0, CHUNK // L, chunk, 0)
    pltpu.sync_copy(fa_v, fa_out.at[pl.ds(base, CHUNK)])


_sc_fgather = functools.partial(
    pl.kernel, _sc_fgather_body,
    out_type=jax.ShapeDtypeStruct((NPAD,), jnp.float32),
    mesh=_MESH,
    compiler_params=pltpu.CompilerParams(needs_layout_passes=False),
    scratch_types=[
        pltpu.VMEM((NPAD,), jnp.float32),
        pltpu.VMEM((CHUNK,), jnp.int32),
        pltpu.VMEM((L,), jnp.int32),
        pltpu.VMEM((CHUNK,), jnp.float32),
    ],
)()


CHUNK2 = NPAD // 16


def _sc_prep_body(nn1_hbm, nn2_hbm, a2_hbm, b2_hbm, sc_hbm, a1_hbm, b1_hbm,
                  smc_out, a2g_out, b2g_out,
                  ja_out, sa_out, a1a_out, b1a_out, a2a_out, b2a_out, kc_out,
                  nn2_v, a2t_v, b2t_v, nn1_v, sc_v, smc_v, a2g_v, b2g_v,
                  sh_smc, sh_a2g, sh_b2g,
                  s_v, a1_v, b1_v, a2_v, b2_v,
                  ja_v, sa_v, a1a_v, b1a_v, a2a_v, b2a_v, kc_v):
    core = lax.axis_index("c")
    sid = lax.axis_index("s")

    @pl.when(core == 0)
    def _():
        base = sid * CHUNK2
        pltpu.sync_copy(nn2_hbm, nn2_v)
        pltpu.sync_copy(a2_hbm, a2t_v)
        pltpu.sync_copy(b2_hbm, b2t_v)
        pltpu.sync_copy(nn1_hbm.at[pl.ds(base, CHUNK2)], nn1_v)
        pltpu.sync_copy(sc_hbm.at[pl.ds(base, CHUNK2)], sc_v)

        def chunk(k, carry):
            sl = pl.ds(k * L, L)
            idx = nn1_v[sl]
            back = plsc.load_gather(nn2_v, [idx])
            iv = base + k * L + lax.iota(jnp.int32, L)
            smc_v[sl] = jnp.where(back != iv, -1.0, sc_v[sl])
            a2g_v[sl] = plsc.load_gather(a2t_v, [idx])
            b2g_v[sl] = plsc.load_gather(b2t_v, [idx])
            return carry

        lax.fori_loop(0, CHUNK2 // L, chunk, 0)
        pltpu.sync_copy(smc_v, smc_out.at[pl.ds(base, CHUNK2)])
        pltpu.sync_copy(a2g_v, a2g_out.at[pl.ds(base, CHUNK2)])
        pltpu.sync_copy(b2g_v, b2g_out.at[pl.ds(base, CHUNK2)])
        pltpu.sync_copy(smc_v, sh_smc.at[pl.ds(base, CHUNK2)])
        pltpu.sync_copy(a2g_v, sh_a2g.at[pl.ds(base, CHUNK2)])
        pltpu.sync_copy(b2g_v, sh_b2g.at[pl.ds(base, CHUNK2)])
        plsc.subcore_barrier()

        @pl.when(sid == 0)
        def _():
            pltpu.sync_copy(sh_smc, s_v)
            pltpu.sync_copy(a1_hbm, a1_v)
            pltpu.sync_copy(b1_hbm, b1_v)
            pltpu.sync_copy(sh_a2g, a2_v)
            pltpu.sync_copy(sh_b2g, b2_v)

            zi = jnp.zeros((L,), jnp.int32)
            zf = jnp.zeros((L,), jnp.float32)

            def fill(k, carry):
                sl = pl.ds(k * L, L)
                ja_v[sl] = zi
                sa_v[sl] = zf - 3.0
                return carry

            lax.fori_loop(0, NPAD // L, fill, 0)

            def compact(k, c):
                sl = pl.ds(k * L, L)
                sv = s_v[sl]
                m = sv >= CONF_BAR
                mi = m.astype(jnp.int32)
                incl = plsc.cumsum(mi)
                slot = c + incl - mi
                iv = k * L + lax.iota(jnp.int32, L)
                plsc.store_scatter(ja_v, [slot], iv, mask=m)
                plsc.store_scatter(sa_v, [slot], sv, mask=m)
                plsc.store_scatter(a1a_v, [slot], a1_v[sl], mask=m)
                plsc.store_scatter(b1a_v, [slot], b1_v[sl], mask=m)
                plsc.store_scatter(a2a_v, [slot], a2_v[sl], mask=m)
                plsc.store_scatter(b2a_v, [slot], b2_v[sl], mask=m)
                return c + jnp.sum(mi)

            kcount = lax.fori_loop(0, NPAD // L, compact, jnp.int32(0))
            kc_v[...] = jnp.zeros((L,), jnp.int32) + kcount
            pltpu.sync_copy(ja_v, ja_out)
            pltpu.sync_copy(sa_v, sa_out)
            pltpu.sync_copy(a1a_v, a1a_out)
            pltpu.sync_copy(b1a_v, b1a_out)
            pltpu.sync_copy(a2a_v, a2a_out)
            pltpu.sync_copy(b2a_v, b2a_out)
            pltpu.sync_copy(kc_v, kc_out)


_sc_prep = functools.partial(
    pl.kernel, _sc_prep_body,
    out_type=[jax.ShapeDtypeStruct((NPAD,), jnp.float32)] * 3
    + [jax.ShapeDtypeStruct((NPAD,), jnp.int32)]
    + [jax.ShapeDtypeStruct((NPAD,), jnp.float32)] * 5
    + [jax.ShapeDtypeStruct((L,), jnp.int32)],
    mesh=_MESH,
    compiler_params=pltpu.CompilerParams(needs_layout_passes=False),
    scratch_types=[
        pltpu.VMEM((NPAD,), jnp.int32),
        pltpu.VMEM((NPAD,), jnp.float32),
        pltpu.VMEM((NPAD,), jnp.float32),
        pltpu.VMEM((CHUNK2,), jnp.int32),
        pltpu.VMEM((CHUNK2,), jnp.float32),
        pltpu.VMEM((CHUNK2,), jnp.float32),
        pltpu.VMEM((CHUNK2,), jnp.float32),
        pltpu.VMEM((CHUNK2,), jnp.float32),
        pltpu.VMEM_SHARED((NPAD,), jnp.float32),
        pltpu.VMEM_SHARED((NPAD,), jnp.float32),
        pltpu.VMEM_SHARED((NPAD,), jnp.float32),
        pltpu.VMEM((NPAD,), jnp.float32),
        pltpu.VMEM((NPAD,), jnp.float32),
        pltpu.VMEM((NPAD,), jnp.float32),
        pltpu.VMEM((NPAD,), jnp.float32),
        pltpu.VMEM((NPAD,), jnp.float32),
        pltpu.VMEM((NPAD,), jnp.int32),
        pltpu.VMEM((NPAD,), jnp.float32),
        pltpu.VMEM((NPAD,), jnp.float32),
        pltpu.VMEM((NPAD,), jnp.float32),
        pltpu.VMEM((NPAD,), jnp.float32),
        pltpu.VMEM((NPAD,), jnp.float32),
        pltpu.VMEM((L,), jnp.int32),
    ],
)()


def _sc_seeds_body(rank_hbm, nn1_hbm, s1_out, s2_out,
                   r_v, n_v, s1_v, s2_v):
    @pl.when(_wid() == 0)
    def _():
        pltpu.sync_copy(rank_hbm, r_v)
        pltpu.sync_copy(nn1_hbm, n_v)

        def chunk(k, carry):
            sl = pl.ds(k * L, L)
            rv = r_v[sl]
            iv = k * L + lax.iota(jnp.int32, L)
            m = (iv < N) & (rv < TOPK)
            plsc.store_scatter(s1_v, [rv], iv, mask=m)
            plsc.store_scatter(s2_v, [rv], n_v[sl], mask=m)
            return carry

        lax.fori_loop(0, NPAD // L, chunk, 0)
        pltpu.sync_copy(s1_v.at[pl.ds(0, TOPK)], s1_out)
        pltpu.sync_copy(s2_v.at[pl.ds(0, TOPK)], s2_out)


_sc_seeds = functools.partial(
    pl.kernel, _sc_seeds_body,
    out_type=[jax.ShapeDtypeStruct((TOPK,), jnp.int32)] * 2,
    mesh=_MESH,
    compiler_params=pltpu.CompilerParams(needs_layout_passes=False),
    scratch_types=[
        pltpu.VMEM((NPAD,), jnp.int32),
        pltpu.VMEM((NPAD,), jnp.int32),
        pltpu.VMEM((NPAD,), jnp.int32),
        pltpu.VMEM((NPAD,), jnp.int32),
    ],
)()


TB = 512
NB = NPAD // TB
_TRI = [(ib, jb) for ib in range(NB) for jb in range(NB) if jb >= ib]
NTRI = len(_TRI)


def _sums_body(ib_ref, jb_ref, a1c, b1c, a2c, b2c, a1r, b1r, a2r, b2r,
               s1_ref, s2_ref):
    t = pl.program_id(0)
    ib = ib_ref[t]
    jb = jb_ref[t]
    ii = ib * TB + jax.lax.broadcasted_iota(jnp.int32, (TB, 1), 0)
    jj = jb * TB + jax.lax.broadcasted_iota(jnp.int32, (1, TB), 1)
    valid = (ii < jj) & (jj < N)
    a1cv, b1cv = a1c[...], b1c[...]
    a2cv, b2cv = a2c[...], b2c[...]
    a1rv, b1rv = a1r[...], b1r[...]
    a2rv, b2rv = a2r[...], b2r[...]
    q1 = (a1cv * a1cv + b1cv * b1cv) + (a1rv * a1rv + b1rv * b1rv) \
        - 2.0 * (a1cv * a1rv + b1cv * b1rv)
    q2 = (a2cv * a2cv + b2cv * b2cv) + (a2rv * a2rv + b2rv * b2rv) \
        - 2.0 * (a2cv * a2rv + b2cv * b2rv)
    d1 = jnp.sqrt(jnp.abs(q1))
    d2 = jnp.sqrt(jnp.abs(q2))
    t1 = jnp.sum(jnp.where(valid, d1, 0.0))
    t2 = jnp.sum(jnp.where(valid, d2, 0.0))

    @pl.when(t == 0)
    def _():
        s1_ref[0, 0] = 0.0
        s2_ref[0, 0] = 0.0

    s1_ref[0, 0] += t1
    s2_ref[0, 0] += t2


def _suppress_body(a1c, b1c, a2c, b2c, sc, a1a, b1a, a2a, b2a, sa,
                   nt, r1sq, r2sq, out_ref):
    i = pl.program_id(0)
    ii = i * TIB + jax.lax.broadcasted_iota(jnp.int32, (TIB, 1), 0)
    ivalid = ii < N
    a1cv, b1cv = a1c[...], b1c[...]
    a2cv, b2cv = a2c[...], b2c[...]
    scv = sc[...]
    r1s = r1sq[0, 0]
    r2s = r2sq[0, 0]

    def body(t, supp):
        a1rv = a1a[t]
        b1rv = b1a[t]
        a2rv = a2a[t]
        b2rv = b2a[t]
        srv = sa[t]
        q1 = jnp.abs((a1cv * a1cv + b1cv * b1cv) + (a1rv * a1rv + b1rv * b1rv)
                     - 2.0 * (a1cv * a1rv + b1cv * b1rv))
        q2 = jnp.abs((a2cv * a2cv + b2cv * b2cv) + (a2rv * a2rv + b2rv * b2rv)
                     - 2.0 * (a2cv * a2rv + b2cv * b2rv))
        close = (q1 < r1s) | (q2 < r2s)
        hit = jnp.any((srv > scv) & close, axis=1, keepdims=True)
        return jnp.maximum(supp, hit.astype(jnp.float32))

    supp_f = jax.lax.fori_loop(0, nt[0, 0], body,
                               jnp.zeros((TIB, 1), dtype=jnp.float32))
    supp = supp_f > 0.0
    final = jnp.where(ivalid & (scv >= CONF_BAR) & ~supp, scv,
                      jnp.where(ivalid, -1.0, -2.0))
    out_ref[...] = final


def _rank_body(fc, fa, ja, nt, out_ref):
    i = pl.program_id(0)
    ii = i * TIB + jax.lax.broadcasted_iota(jnp.int32, (TIB, 1), 0)
    fcv = fc[...]

    def body(t, carry):
        c_gt, c_posb, p = carry
        frv = fa[t]
        jv = ja[t]
        pos_row = frv > -1.0
        before = jv < ii
        gt = (frv > fcv) | ((frv == fcv) & before)
        c_gt = c_gt + jnp.sum(gt.astype(jnp.int32), axis=1, keepdims=True)
        c_posb = c_posb + jnp.sum((pos_row & before).astype(jnp.int32),
                                  axis=1, keepdims=True)
        p = p + jnp.sum(pos_row.astype(jnp.int32))
        return c_gt, c_posb, p

    zero = jnp.zeros((TIB, 1), dtype=jnp.int32)
    c_gt, c_posb, p = jax.lax.fori_loop(0, nt[0, 0], body,
                                        (zero, zero, jnp.int32(0)))
    rank = jnp.where(fcv > -1.0, c_gt, p + ii - c_posb)
    out_ref[...] = rank


def _col_spec():
    return pl.BlockSpec((TIB, 1), lambda i: (i, 0))


def _row_spec():
    return pl.BlockSpec((1, NPAD), lambda i: (0, 0))


def _act_spec():
    return pl.BlockSpec((NJT, 1, TJ), lambda i: (0, 0, 0))


def _one_spec():
    return pl.BlockSpec((1, 1), lambda i: (0, 0), memory_space=pltpu.SMEM)


def kernel(nn_index1, nn_index2, x1, x2, match_score, topk):
    del topk
    pad = NPAD - N

    def padv(v, val):
        return jnp.pad(v, (0, pad), constant_values=val)

    nn1p = padv(nn_index1[0], 0)
    nn2p = padv(nn_index2[0], 0)
    a1 = padv(x1[0, :, 0], 0.0)
    b1 = padv(x1[0, :, 1], 0.0)
    a2t = padv(x2[0, :, 0], 0.0)
    b2t = padv(x2[0, :, 1], 0.0)
    scp = padv(match_score[0], -3.0)

    (s, a2, b2, ja_v, sa_v, a1a_v, b1a_v, a2a_v, b2a_v, kc) = _sc_prep(
        nn1p, nn2p, a2t, b2t, scp, a1, b1)

    col = lambda v: v.reshape(NPAD, 1)
    act = lambda v: v.reshape(NJT, 1, TJ)

    ibs = jnp.array([p[0] for p in _TRI], jnp.int32)
    jbs = jnp.array([p[1] for p in _TRI], jnp.int32)
    grid_spec = pltpu.PrefetchScalarGridSpec(
        num_scalar_prefetch=2,
        grid=(NTRI,),
        in_specs=[pl.BlockSpec((TB, 1), lambda t, ib, jb: (ib[t], 0))] * 4
        + [pl.BlockSpec((1, TB), lambda t, ib, jb: (0, jb[t]))] * 4,
        out_specs=[
            pl.BlockSpec((1, 1), lambda t, ib, jb: (0, 0),
                         memory_space=pltpu.SMEM),
            pl.BlockSpec((1, 1), lambda t, ib, jb: (0, 0),
                         memory_space=pltpu.SMEM),
        ],
    )
    s1, s2 = pl.pallas_call(
        _sums_body,
        grid_spec=grid_spec,
        out_shape=[jax.ShapeDtypeStruct((1, 1), jnp.float32)] * 2,
    )(ibs, jbs, col(a1), col(b1), col(a2), col(b2),
      a1.reshape(1, NPAD), b1.reshape(1, NPAD),
      a2.reshape(1, NPAD), b2.reshape(1, NPAD))

    scale = 2.0 * NMS_RADIUS / (N * N)
    r1 = s1[0, 0] * scale
    r2 = s2[0, 0] * scale
    r1sq = (r1 * r1).reshape(1, 1)
    r2sq = (r2 * r2).reshape(1, 1)
    kcount = kc[0]
    nt = ((kcount + TJ - 1) // TJ).reshape(1, 1)

    grid = NPAD // TIB
    fcol = pl.pallas_call(
        _suppress_body,
        grid=(grid,),
        in_specs=[_col_spec()] * 5 + [_act_spec()] * 5
        + [_one_spec(), _one_spec(), _one_spec()],
        out_specs=_col_spec(),
        out_shape=jax.ShapeDtypeStruct((NPAD, 1), jnp.float32),
    )(col(a1), col(b1), col(a2), col(b2), col(s),
      act(a1a_v), act(b1a_v), act(a2a_v), act(b2a_v), act(sa_v),
      nt, r1sq, r2sq)

    fa_v = _sc_fgather(fcol.reshape(NPAD), ja_v, kc)

    rank = pl.pallas_call(
        _rank_body,
        grid=(grid,),
        in_specs=[_col_spec(), _act_spec(), _act_spec(), _one_spec()],
        out_specs=_col_spec(),
        out_shape=jax.ShapeDtypeStruct((NPAD, 1), jnp.int32),
    )(fcol, act(fa_v), act(ja_v), nt)

    seed1, seed2 = _sc_seeds(rank.reshape(NPAD), nn1p)

    return seed1[None, :], seed2[None, :], fcol[:N, 0][None, :]


# speedup vs baseline: 1.0748x; 1.0748x over previous
import functools

import jax
import jax.numpy as jnp
from jax import lax
from jax.experimental import pallas as pl
from jax.experimental.pallas import tpu as pltpu
from jax.experimental.pallas import tpu_sc as plsc

N = 5000
NPAD = 5120
TIB = 512
TJ = 512
NJT = NPAD // TJ
CONF_BAR = 0.1
NMS_RADIUS = 0.01
TOPK = 512
NW = 32
CHUNK = NPAD // NW
L = 16

_MESH = plsc.VectorSubcoreMesh(core_axis_name="c", subcore_axis_name="s")


def _wid():
    return lax.axis_index("s") * 2 + lax.axis_index("c")


def _sc_gather_body(nn1_hbm, nn2_hbm, a2_hbm, b2_hbm, sc_hbm,
                    smc_out, a2g_out, b2g_out,
                    nn2_v, a2_v, b2_v, nn1_v, sc_v, smc_v, a2g_v, b2g_v):
    wid = _wid()
    base = wid * CHUNK
    pltpu.sync_copy(nn2_hbm, nn2_v)
    pltpu.sync_copy(a2_hbm, a2_v)
    pltpu.sync_copy(b2_hbm, b2_v)
    pltpu.sync_copy(nn1_hbm.at[pl.ds(base, CHUNK)], nn1_v)
    pltpu.sync_copy(sc_hbm.at[pl.ds(base, CHUNK)], sc_v)

    def chunk(k, carry):
        sl = pl.ds(k * L, L)
        idx = nn1_v[sl]
        back = plsc.load_gather(nn2_v, [idx])
        iv = base + k * L + lax.iota(jnp.int32, L)
        smc_v[sl] = jnp.where(back != iv, -1.0, sc_v[sl])
        a2g_v[sl] = plsc.load_gather(a2_v, [idx])
        b2g_v[sl] = plsc.load_gather(b2_v, [idx])
        return carry

    lax.fori_loop(0, CHUNK // L, chunk, 0)
    pltpu.sync_copy(smc_v, smc_out.at[pl.ds(base, CHUNK)])
    pltpu.sync_copy(a2g_v, a2g_out.at[pl.ds(base, CHUNK)])
    pltpu.sync_copy(b2g_v, b2g_out.at[pl.ds(base, CHUNK)])


_sc_gather = functools.partial(
    pl.kernel, _sc_gather_body,
    out_type=[jax.ShapeDtypeStruct((NPAD,), jnp.float32)] * 3,
    mesh=_MESH,
    compiler_params=pltpu.CompilerParams(needs_layout_passes=False),
    scratch_types=[
        pltpu.VMEM((N,), jnp.int32),
        pltpu.VMEM((N,), jnp.float32),
        pltpu.VMEM((N,), jnp.float32),
        pltpu.VMEM((CHUNK,), jnp.int32),
        pltpu.VMEM((CHUNK,), jnp.float32),
        pltpu.VMEM((CHUNK,), jnp.float32),
        pltpu.VMEM((CHUNK,), jnp.float32),
        pltpu.VMEM((CHUNK,), jnp.float32),
    ],
)()


def _sc_compact_body(smc_hbm, a1_hbm, b1_hbm, a2g_hbm, b2g_hbm,
                     ja_out, sa_out, a1a_out, b1a_out, a2a_out, b2a_out,
                     kc_out,
                     s_v, a1_v, b1_v, a2_v, b2_v,
                     ja_v, sa_v, a1a_v, b1a_v, a2a_v, b2a_v, kc_v):
    @pl.when(_wid() == 0)
    def _():
        pltpu.sync_copy(smc_hbm, s_v)
        pltpu.sync_copy(a1_hbm, a1_v)
        pltpu.sync_copy(b1_hbm, b1_v)
        pltpu.sync_copy(a2g_hbm, a2_v)
        pltpu.sync_copy(b2g_hbm, b2_v)

        zi = jnp.zeros((L,), jnp.int32)
        zf = jnp.zeros((L,), jnp.float32)

        def fill(k, carry):
            sl = pl.ds(k * L, L)
            ja_v[sl] = zi
            sa_v[sl] = zf - 3.0
            a1a_v[sl] = zf
            return carry

        lax.fori_loop(0, NPAD // L, fill, 0)

        def compact(k, c):
            sl = pl.ds(k * L, L)
            sv = s_v[sl]
            m = sv >= CONF_BAR
            mi = m.astype(jnp.int32)
            incl = plsc.cumsum(mi)
            slot = c + incl - mi
            iv = k * L + lax.iota(jnp.int32, L)
            plsc.store_scatter(ja_v, [slot], iv, mask=m)
            plsc.store_scatter(sa_v, [slot], sv, mask=m)
            plsc.store_scatter(a1a_v, [slot], a1_v[sl], mask=m)
            plsc.store_scatter(b1a_v, [slot], b1_v[sl], mask=m)
            plsc.store_scatter(a2a_v, [slot], a2_v[sl], mask=m)
            plsc.store_scatter(b2a_v, [slot], b2_v[sl], mask=m)
            return c + jnp.sum(mi)

        kcount = lax.fori_loop(0, NPAD // L, compact, jnp.int32(0))
        kc_v[...] = jnp.zeros((L,), jnp.int32) + kcount
        pltpu.sync_copy(ja_v, ja_out)
        pltpu.sync_copy(sa_v, sa_out)
        pltpu.sync_copy(a1a_v, a1a_out)
        pltpu.sync_copy(b1a_v, b1a_out)
        pltpu.sync_copy(a2a_v, a2a_out)
        pltpu.sync_copy(b2a_v, b2a_out)
        pltpu.sync_copy(kc_v, kc_out)


_sc_compact = functools.partial(
    pl.kernel, _sc_compact_body,
    out_type=[jax.ShapeDtypeStruct((NPAD,), jnp.int32)]
    + [jax.ShapeDtypeStruct((NPAD,), jnp.float32)] * 5
    + [jax.ShapeDtypeStruct((L,), jnp.int32)],
    mesh=_MESH,
    compiler_params=pltpu.CompilerParams(needs_layout_passes=False),
    scratch_types=[
        pltpu.VMEM((NPAD,), jnp.float32),
        pltpu.VMEM((NPAD,), jnp.float32),
        pltpu.VMEM((NPAD,), jnp.float32),
        pltpu.VMEM((NPAD,), jnp.float32),
        pltpu.VMEM((NPAD,), jnp.float32),
        pltpu.VMEM((NPAD,), jnp.int32),
        pltpu.VMEM((NPAD,), jnp.float32),
        pltpu.VMEM((NPAD,), jnp.float32),
        pltpu.VMEM((NPAD,), jnp.float32),
        pltpu.VMEM((NPAD,), jnp.float32),
        pltpu.VMEM((NPAD,), jnp.float32),
        pltpu.VMEM((L,), jnp.int32),
    ],
)()


def _sc_fgather_body(f_hbm, ja_hbm, kc_hbm, fa_out,
                     f_v, ja_v, kc_v, fa_v):
    wid = _wid()
    base = wid * CHUNK
    pltpu.sync_copy(f_hbm, f_v)
    pltpu.sync_copy(ja_hbm.at[pl.ds(base, CHUNK)], ja_v)
    pltpu.sync_copy(kc_hbm, kc_v)
    kvec = kc_v[...]

    def chunk(k, carry):
        sl = pl.ds(k * L, L)
        jv = ja_v[sl]
        kg = base + k * L + lax.iota(jnp.int32, L)
        m = kg < kvec
        vals = plsc.load_gather(f_v, [jnp.where(m, jv, 0)])
        fa_v[sl] = jnp.where(m, vals, -3.0)
        return carry

    lax.fori_loop(0, CHUNK // L, chunk, 0)
    pltpu.sync_copy(fa_v, fa_out.at[pl.ds(base, CHUNK)])


_sc_fgather = functools.partial(
    pl.kernel, _sc_fgather_body,
    out_type=jax.ShapeDtypeStruct((NPAD,), jnp.float32),
    mesh=_MESH,
    compiler_params=pltpu.CompilerParams(needs_layout_passes=False),
    scratch_types=[
        pltpu.VMEM((NPAD,), jnp.float32),
        pltpu.VMEM((CHUNK,), jnp.int32),
        pltpu.VMEM((L,), jnp.int32),
        pltpu.VMEM((CHUNK,), jnp.float32),
    ],
)()


def _sc_seeds_body(rank_hbm, nn1_hbm, s1_out, s2_out,
                   r_v, n_v, s1_v, s2_v):
    @pl.when(_wid() == 0)
    def _():
        pltpu.sync_copy(rank_hbm, r_v)
        pltpu.sync_copy(nn1_hbm, n_v)

        def chunk(k, carry):
            sl = pl.ds(k * L, L)
            rv = r_v[sl]
            iv = k * L + lax.iota(jnp.int32, L)
            m = (iv < N) & (rv < TOPK)
            plsc.store_scatter(s1_v, [rv], iv, mask=m)
            plsc.store_scatter(s2_v, [rv], n_v[sl], mask=m)
            return carry

        lax.fori_loop(0, NPAD // L, chunk, 0)
        pltpu.sync_copy(s1_v.at[pl.ds(0, TOPK)], s1_out)
        pltpu.sync_copy(s2_v.at[pl.ds(0, TOPK)], s2_out)


_sc_seeds = functools.partial(
    pl.kernel, _sc_seeds_body,
    out_type=[jax.ShapeDtypeStruct((TOPK,), jnp.int32)] * 2,
    mesh=_MESH,
    compiler_params=pltpu.CompilerParams(needs_layout_passes=False),
    scratch_types=[
        pltpu.VMEM((NPAD,), jnp.int32),
        pltpu.VMEM((NPAD,), jnp.int32),
        pltpu.VMEM((NPAD,), jnp.int32),
        pltpu.VMEM((NPAD,), jnp.int32),
    ],
)()


TB = 512
NB = NPAD // TB
_TRI = [(ib, jb) for ib in range(NB) for jb in range(NB) if jb >= ib]
NTRI = len(_TRI)


def _sums_body(ib_ref, jb_ref, a1c, b1c, a2c, b2c, a1r, b1r, a2r, b2r,
               s1_ref, s2_ref):
    t = pl.program_id(0)
    ib = ib_ref[t]
    jb = jb_ref[t]
    ii = ib * TB + jax.lax.broadcasted_iota(jnp.int32, (TB, 1), 0)
    jj = jb * TB + jax.lax.broadcasted_iota(jnp.int32, (1, TB), 1)
    valid = (ii < jj) & (jj < N)
    a1cv, b1cv = a1c[...], b1c[...]
    a2cv, b2cv = a2c[...], b2c[...]
    a1rv, b1rv = a1r[...], b1r[...]
    a2rv, b2rv = a2r[...], b2r[...]
    q1 = (a1cv * a1cv + b1cv * b1cv) + (a1rv * a1rv + b1rv * b1rv) \
        - 2.0 * (a1cv * a1rv + b1cv * b1rv)
    q2 = (a2cv * a2cv + b2cv * b2cv) + (a2rv * a2rv + b2rv * b2rv) \
        - 2.0 * (a2cv * a2rv + b2cv * b2rv)
    d1 = jnp.sqrt(jnp.abs(q1))
    d2 = jnp.sqrt(jnp.abs(q2))
    t1 = jnp.sum(jnp.where(valid, d1, 0.0))
    t2 = jnp.sum(jnp.where(valid, d2, 0.0))

    @pl.when(t == 0)
    def _():
        s1_ref[0, 0] = 0.0
        s2_ref[0, 0] = 0.0

    s1_ref[0, 0] += t1
    s2_ref[0, 0] += t2


def _suppress_body(a1c, b1c, a2c, b2c, sc, a1a, b1a, a2a, b2a, sa,
                   nt, r1sq, r2sq, out_ref):
    i = pl.program_id(0)
    ii = i * TIB + jax.lax.broadcasted_iota(jnp.int32, (TIB, 1), 0)
    ivalid = ii < N
    a1cv, b1cv = a1c[...], b1c[...]
    a2cv, b2cv = a2c[...], b2c[...]
    scv = sc[...]
    r1s = r1sq[0, 0]
    r2s = r2sq[0, 0]

    def body(t, supp):
        a1rv = a1a[t]
        b1rv = b1a[t]
        a2rv = a2a[t]
        b2rv = b2a[t]
        srv = sa[t]
        q1 = jnp.abs((a1cv * a1cv + b1cv * b1cv) + (a1rv * a1rv + b1rv * b1rv)
                     - 2.0 * (a1cv * a1rv + b1cv * b1rv))
        q2 = jnp.abs((a2cv * a2cv + b2cv * b2cv) + (a2rv * a2rv + b2rv * b2rv)
                     - 2.0 * (a2cv * a2rv + b2cv * b2rv))
        close = (q1 < r1s) | (q2 < r2s)
        hit = jnp.any((srv > scv) & close, axis=1, keepdims=True)
        return jnp.maximum(supp, hit.astype(jnp.float32))

    supp_f = jax.lax.fori_loop(0, nt[0, 0], body,
                               jnp.zeros((TIB, 1), dtype=jnp.float32))
    supp = supp_f > 0.0
    final = jnp.where(ivalid & (scv >= CONF_BAR) & ~supp, scv,
                      jnp.where(ivalid, -1.0, -2.0))
    out_ref[...] = final


def _rank_body(fc, fa, ja, nt, out_ref):
    i = pl.program_id(0)
    ii = i * TIB + jax.lax.broadcasted_iota(jnp.int32, (TIB, 1), 0)
    fcv = fc[...]

    def body(t, carry):
        c_gt, c_posb, p = carry
        frv = fa[t]
        jv = ja[t]
        pos_row = frv > -1.0
        before = jv < ii
        gt = (frv > fcv) | ((frv == fcv) & before)
        c_gt = c_gt + jnp.sum(gt.astype(jnp.int32), axis=1, keepdims=True)
        c_posb = c_posb + jnp.sum((pos_row & before).astype(jnp.int32),
                                  axis=1, keepdims=True)
        p = p + jnp.sum(pos_row.astype(jnp.int32))
        return c_gt, c_posb, p

    zero = jnp.zeros((TIB, 1), dtype=jnp.int32)
    c_gt, c_posb, p = jax.lax.fori_loop(0, nt[0, 0], body,
                                        (zero, zero, jnp.int32(0)))
    rank = jnp.where(fcv > -1.0, c_gt, p + ii - c_posb)
    out_ref[...] = rank


def _col_spec():
    return pl.BlockSpec((TIB, 1), lambda i: (i, 0))


def _row_spec():
    return pl.BlockSpec((1, NPAD), lambda i: (0, 0))


def _act_spec():
    return pl.BlockSpec((NJT, 1, TJ), lambda i: (0, 0, 0))


def _one_spec():
    return pl.BlockSpec((1, 1), lambda i: (0, 0), memory_space=pltpu.SMEM)


def kernel(nn_index1, nn_index2, x1, x2, match_score, topk):
    del topk
    pad = NPAD - N

    def padv(v, val):
        return jnp.pad(v, (0, pad), constant_values=val)

    nn1p = padv(nn_index1[0], 0)
    a1 = padv(x1[0, :, 0], 0.0)
    b1 = padv(x1[0, :, 1], 0.0)
    scp = padv(match_score[0], -3.0)

    s, a2, b2 = _sc_gather(nn1p, nn_index2[0], x2[0, :, 0], x2[0, :, 1],
                           scp)
    ja_v, sa_v, a1a_v, b1a_v, a2a_v, b2a_v, kc = _sc_compact(
        s, a1, b1, a2, b2)

    col = lambda v: v.reshape(NPAD, 1)
    act = lambda v: v.reshape(NJT, 1, TJ)

    ibs = jnp.array([p[0] for p in _TRI], jnp.int32)
    jbs = jnp.array([p[1] for p in _TRI], jnp.int32)
    grid_spec = pltpu.PrefetchScalarGridSpec(
        num_scalar_prefetch=2,
        grid=(NTRI,),
        in_specs=[pl.BlockSpec((TB, 1), lambda t, ib, jb: (ib[t], 0))] * 4
        + [pl.BlockSpec((1, TB), lambda t, ib, jb: (0, jb[t]))] * 4,
        out_specs=[
            pl.BlockSpec((1, 1), lambda t, ib, jb: (0, 0),
                         memory_space=pltpu.SMEM),
            pl.BlockSpec((1, 1), lambda t, ib, jb: (0, 0),
                         memory_space=pltpu.SMEM),
        ],
    )
    s1, s2 = pl.pallas_call(
        _sums_body,
        grid_spec=grid_spec,
        out_shape=[jax.ShapeDtypeStruct((1, 1), jnp.float32)] * 2,
    )(ibs, jbs, col(a1), col(b1), col(a2), col(b2),
      a1.reshape(1, NPAD), b1.reshape(1, NPAD),
      a2.reshape(1, NPAD), b2.reshape(1, NPAD))

    scale = 2.0 * NMS_RADIUS / (N * N)
    r1 = s1[0, 0] * scale
    r2 = s2[0, 0] * scale
    r1sq = (r1 * r1).reshape(1, 1)
    r2sq = (r2 * r2).reshape(1, 1)
    kcount = kc[0]
    nt = ((kcount + TJ - 1) // TJ).reshape(1, 1)

    grid = NPAD // TIB
    fcol = pl.pallas_call(
        _suppress_body,
        grid=(grid,),
        in_specs=[_col_spec()] * 5 + [_act_spec()] * 5
        + [_one_spec(), _one_spec(), _one_spec()],
        out_specs=_col_spec(),
        out_shape=jax.ShapeDtypeStruct((NPAD, 1), jnp.float32),
    )(col(a1), col(b1), col(a2), col(b2), col(s),
      act(a1a_v), act(b1a_v), act(a2a_v), act(b2a_v), act(sa_v),
      nt, r1sq, r2sq)

    fa_v = _sc_fgather(fcol.reshape(NPAD), ja_v, kc)

    rank = pl.pallas_call(
        _rank_body,
        grid=(grid,),
        in_specs=[_col_spec(), _act_spec(), _act_spec(), _one_spec()],
        out_specs=_col_spec(),
        out_shape=jax.ShapeDtypeStruct((NPAD, 1), jnp.int32),
    )(fcol, act(fa_v), act(ja_v), nt)

    seed1, seed2 = _sc_seeds(rank.reshape(NPAD), nn1p)

    return seed1[None, :], seed2[None, :], fcol[:N, 0][None, :]


# speedup vs baseline: 1.0756x; 1.0007x over previous
import functools

import jax
import jax.numpy as jnp
from jax import lax
from jax.experimental import pallas as pl
from jax.experimental.pallas import tpu as pltpu
from jax.experimental.pallas import tpu_sc as plsc

N = 5000
NPAD = 5120
TIB = 1024
TJ = 512
NJT = NPAD // TJ
CONF_BAR = 0.1
NMS_RADIUS = 0.01
TOPK = 512
NW = 32
CHUNK = NPAD // NW
L = 16

_MESH = plsc.VectorSubcoreMesh(core_axis_name="c", subcore_axis_name="s")


def _wid():
    return lax.axis_index("s") * 2 + lax.axis_index("c")


def _sc_gather_body(nn1_hbm, nn2_hbm, a2_hbm, b2_hbm, sc_hbm,
                    smc_out, a2g_out, b2g_out,
                    nn2_v, a2_v, b2_v, nn1_v, sc_v, smc_v, a2g_v, b2g_v):
    wid = _wid()
    base = wid * CHUNK
    pltpu.sync_copy(nn2_hbm, nn2_v)
    pltpu.sync_copy(a2_hbm, a2_v)
    pltpu.sync_copy(b2_hbm, b2_v)
    pltpu.sync_copy(nn1_hbm.at[pl.ds(base, CHUNK)], nn1_v)
    pltpu.sync_copy(sc_hbm.at[pl.ds(base, CHUNK)], sc_v)

    def chunk(k, carry):
        sl = pl.ds(k * L, L)
        idx = nn1_v[sl]
        back = plsc.load_gather(nn2_v, [idx])
        iv = base + k * L + lax.iota(jnp.int32, L)
        smc_v[sl] = jnp.where(back != iv, -1.0, sc_v[sl])
        a2g_v[sl] = plsc.load_gather(a2_v, [idx])
        b2g_v[sl] = plsc.load_gather(b2_v, [idx])
        return carry

    lax.fori_loop(0, CHUNK // L, chunk, 0)
    pltpu.sync_copy(smc_v, smc_out.at[pl.ds(base, CHUNK)])
    pltpu.sync_copy(a2g_v, a2g_out.at[pl.ds(base, CHUNK)])
    pltpu.sync_copy(b2g_v, b2g_out.at[pl.ds(base, CHUNK)])


_sc_gather = functools.partial(
    pl.kernel, _sc_gather_body,
    out_type=[jax.ShapeDtypeStruct((NPAD,), jnp.float32)] * 3,
    mesh=_MESH,
    compiler_params=pltpu.CompilerParams(needs_layout_passes=False),
    scratch_types=[
        pltpu.VMEM((N,), jnp.int32),
        pltpu.VMEM((N,), jnp.float32),
        pltpu.VMEM((N,), jnp.float32),
        pltpu.VMEM((CHUNK,), jnp.int32),
        pltpu.VMEM((CHUNK,), jnp.float32),
        pltpu.VMEM((CHUNK,), jnp.float32),
        pltpu.VMEM((CHUNK,), jnp.float32),
        pltpu.VMEM((CHUNK,), jnp.float32),
    ],
)()


def _sc_compact_body(smc_hbm, a1_hbm, b1_hbm, a2g_hbm, b2g_hbm,
                     ja_out, sa_out, a1a_out, b1a_out, a2a_out, b2a_out,
                     kc_out,
                     s_v, a1_v, b1_v, a2_v, b2_v,
                     ja_v, sa_v, a1a_v, b1a_v, a2a_v, b2a_v, kc_v):
    @pl.when(_wid() == 0)
    def _():
        pltpu.sync_copy(smc_hbm, s_v)
        pltpu.sync_copy(a1_hbm, a1_v)
        pltpu.sync_copy(b1_hbm, b1_v)
        pltpu.sync_copy(a2g_hbm, a2_v)
        pltpu.sync_copy(b2g_hbm, b2_v)

        zi = jnp.zeros((L,), jnp.int32)
        zf = jnp.zeros((L,), jnp.float32)

        def fill(k, carry):
            sl = pl.ds(k * L, L)
            ja_v[sl] = zi
            sa_v[sl] = zf - 3.0
            a1a_v[sl] = zf
            return carry

        lax.fori_loop(0, NPAD // L, fill, 0)

        def compact(k, c):
            sl = pl.ds(k * L, L)
            sv = s_v[sl]
            m = sv >= CONF_BAR
            mi = m.astype(jnp.int32)
            incl = plsc.cumsum(mi)
            slot = c + incl - mi
            iv = k * L + lax.iota(jnp.int32, L)
            plsc.store_scatter(ja_v, [slot], iv, mask=m)
            plsc.store_scatter(sa_v, [slot], sv, mask=m)
            plsc.store_scatter(a1a_v, [slot], a1_v[sl], mask=m)
            plsc.store_scatter(b1a_v, [slot], b1_v[sl], mask=m)
            plsc.store_scatter(a2a_v, [slot], a2_v[sl], mask=m)
            plsc.store_scatter(b2a_v, [slot], b2_v[sl], mask=m)
            return c + jnp.sum(mi)

        kcount = lax.fori_loop(0, NPAD // L, compact, jnp.int32(0))
        kc_v[...] = jnp.zeros((L,), jnp.int32) + kcount
        pltpu.sync_copy(ja_v, ja_out)
        pltpu.sync_copy(sa_v, sa_out)
        pltpu.sync_copy(a1a_v, a1a_out)
        pltpu.sync_copy(b1a_v, b1a_out)
        pltpu.sync_copy(a2a_v, a2a_out)
        pltpu.sync_copy(b2a_v, b2a_out)
        pltpu.sync_copy(kc_v, kc_out)


_sc_compact = functools.partial(
    pl.kernel, _sc_compact_body,
    out_type=[jax.ShapeDtypeStruct((NPAD,), jnp.int32)]
    + [jax.ShapeDtypeStruct((NPAD,), jnp.float32)] * 5
    + [jax.ShapeDtypeStruct((L,), jnp.int32)],
    mesh=_MESH,
    compiler_params=pltpu.CompilerParams(needs_layout_passes=False),
    scratch_types=[
        pltpu.VMEM((NPAD,), jnp.float32),
        pltpu.VMEM((NPAD,), jnp.float32),
        pltpu.VMEM((NPAD,), jnp.float32),
        pltpu.VMEM((NPAD,), jnp.float32),
        pltpu.VMEM((NPAD,), jnp.float32),
        pltpu.VMEM((NPAD,), jnp.int32),
        pltpu.VMEM((NPAD,), jnp.float32),
        pltpu.VMEM((NPAD,), jnp.float32),
        pltpu.VMEM((NPAD,), jnp.float32),
        pltpu.VMEM((NPAD,), jnp.float32),
        pltpu.VMEM((NPAD,), jnp.float32),
        pltpu.VMEM((L,), jnp.int32),
    ],
)()


def _sc_fgather_body(f_hbm, ja_hbm, kc_hbm, fa_out,
                     f_v, ja_v, kc_v, fa_v):
    wid = _wid()
    base = wid * CHUNK
    pltpu.sync_copy(f_hbm, f_v)
    pltpu.sync_copy(ja_hbm.at[pl.ds(base, CHUNK)], ja_v)
    pltpu.sync_copy(kc_hbm, kc_v)
    kvec = kc_v[...]

    def chunk(k, carry):
        sl = pl.ds(k * L, L)
        jv = ja_v[sl]
        kg = base + k * L + lax.iota(jnp.int32, L)
        m = kg < kvec
        vals = plsc.load_gather(f_v, [jnp.where(m, jv, 0)])
        fa_v[sl] = jnp.where(m, vals, -3.0)
        return carry

    lax.fori_loop(0, CHUNK // L, chunk, 0)
    pltpu.sync_copy(fa_v, fa_out.at[pl.ds(base, CHUNK)])


_sc_fgather = functools.partial(
    pl.kernel, _sc_fgather_body,
    out_type=jax.ShapeDtypeStruct((NPAD,), jnp.float32),
    mesh=_MESH,
    compiler_params=pltpu.CompilerParams(needs_layout_passes=False),
    scratch_types=[
        pltpu.VMEM((NPAD,), jnp.float32),
        pltpu.VMEM((CHUNK,), jnp.int32),
        pltpu.VMEM((L,), jnp.int32),
        pltpu.VMEM((CHUNK,), jnp.float32),
    ],
)()


def _sc_seeds_body(rank_hbm, nn1_hbm, s1_out, s2_out,
                   r_v, n_v, s1_v, s2_v):
    @pl.when(_wid() == 0)
    def _():
        pltpu.sync_copy(rank_hbm, r_v)
        pltpu.sync_copy(nn1_hbm, n_v)

        def chunk(k, carry):
            sl = pl.ds(k * L, L)
            rv = r_v[sl]
            iv = k * L + lax.iota(jnp.int32, L)
            m = (iv < N) & (rv < TOPK)
            plsc.store_scatter(s1_v, [rv], iv, mask=m)
            plsc.store_scatter(s2_v, [rv], n_v[sl], mask=m)
            return carry

        lax.fori_loop(0, NPAD // L, chunk, 0)
        pltpu.sync_copy(s1_v.at[pl.ds(0, TOPK)], s1_out)
        pltpu.sync_copy(s2_v.at[pl.ds(0, TOPK)], s2_out)


_sc_seeds = functools.partial(
    pl.kernel, _sc_seeds_body,
    out_type=[jax.ShapeDtypeStruct((TOPK,), jnp.int32)] * 2,
    mesh=_MESH,
    compiler_params=pltpu.CompilerParams(needs_layout_passes=False),
    scratch_types=[
        pltpu.VMEM((NPAD,), jnp.int32),
        pltpu.VMEM((NPAD,), jnp.int32),
        pltpu.VMEM((NPAD,), jnp.int32),
        pltpu.VMEM((NPAD,), jnp.int32),
    ],
)()


TB = 512
NB = NPAD // TB
_TRI = [(ib, jb) for ib in range(NB) for jb in range(NB) if jb >= ib]
NTRI = len(_TRI)


def _sums_body(ib_ref, jb_ref, a1c, b1c, a2c, b2c, a1r, b1r, a2r, b2r,
               s1_ref, s2_ref):
    t = pl.program_id(0)
    ib = ib_ref[t]
    jb = jb_ref[t]
    ii = ib * TB + jax.lax.broadcasted_iota(jnp.int32, (TB, 1), 0)
    jj = jb * TB + jax.lax.broadcasted_iota(jnp.int32, (1, TB), 1)
    valid = (ii < jj) & (jj < N)
    a1cv, b1cv = a1c[...], b1c[...]
    a2cv, b2cv = a2c[...], b2c[...]
    a1rv, b1rv = a1r[...], b1r[...]
    a2rv, b2rv = a2r[...], b2r[...]
    q1 = (a1cv * a1cv + b1cv * b1cv) + (a1rv * a1rv + b1rv * b1rv) \
        - 2.0 * (a1cv * a1rv + b1cv * b1rv)
    q2 = (a2cv * a2cv + b2cv * b2cv) + (a2rv * a2rv + b2rv * b2rv) \
        - 2.0 * (a2cv * a2rv + b2cv * b2rv)
    d1 = jnp.sqrt(jnp.abs(q1))
    d2 = jnp.sqrt(jnp.abs(q2))
    t1 = jnp.sum(jnp.where(valid, d1, 0.0))
    t2 = jnp.sum(jnp.where(valid, d2, 0.0))

    @pl.when(t == 0)
    def _():
        s1_ref[0, 0] = 0.0
        s2_ref[0, 0] = 0.0

    s1_ref[0, 0] += t1
    s2_ref[0, 0] += t2


def _suppress_body(a1c, b1c, a2c, b2c, sc, a1a, b1a, a2a, b2a, sa,
                   nt, r1sq, r2sq, out_ref):
    i = pl.program_id(0)
    ii = i * TIB + jax.lax.broadcasted_iota(jnp.int32, (TIB, 1), 0)
    ivalid = ii < N
    a1cv, b1cv = a1c[...], b1c[...]
    a2cv, b2cv = a2c[...], b2c[...]
    scv = sc[...]
    r1s = r1sq[0, 0]
    r2s = r2sq[0, 0]

    def body(t, supp):
        a1rv = a1a[t]
        b1rv = b1a[t]
        a2rv = a2a[t]
        b2rv = b2a[t]
        srv = sa[t]
        q1 = jnp.abs((a1cv * a1cv + b1cv * b1cv) + (a1rv * a1rv + b1rv * b1rv)
                     - 2.0 * (a1cv * a1rv + b1cv * b1rv))
        q2 = jnp.abs((a2cv * a2cv + b2cv * b2cv) + (a2rv * a2rv + b2rv * b2rv)
                     - 2.0 * (a2cv * a2rv + b2cv * b2rv))
        close = (q1 < r1s) | (q2 < r2s)
        hit = jnp.any((srv > scv) & close, axis=1, keepdims=True)
        return jnp.maximum(supp, hit.astype(jnp.float32))

    supp_f = jax.lax.fori_loop(0, nt[0, 0], body,
                               jnp.zeros((TIB, 1), dtype=jnp.float32))
    supp = supp_f > 0.0
    final = jnp.where(ivalid & (scv >= CONF_BAR) & ~supp, scv,
                      jnp.where(ivalid, -1.0, -2.0))
    out_ref[...] = final


def _rank_body(fc, fa, ja, nt, out_ref):
    i = pl.program_id(0)
    ii = i * TIB + jax.lax.broadcasted_iota(jnp.int32, (TIB, 1), 0)
    fcv = fc[...]

    def body(t, carry):
        c_gt, c_posb, p = carry
        frv = fa[t]
        jv = ja[t]
        pos_row = frv > -1.0
        before = jv < ii
        gt = (frv > fcv) | ((frv == fcv) & before)
        c_gt = c_gt + jnp.sum(gt.astype(jnp.int32), axis=1, keepdims=True)
        c_posb = c_posb + jnp.sum((pos_row & before).astype(jnp.int32),
                                  axis=1, keepdims=True)
        p = p + jnp.sum(pos_row.astype(jnp.int32))
        return c_gt, c_posb, p

    zero = jnp.zeros((TIB, 1), dtype=jnp.int32)
    c_gt, c_posb, p = jax.lax.fori_loop(0, nt[0, 0], body,
                                        (zero, zero, jnp.int32(0)))
    rank = jnp.where(fcv > -1.0, c_gt, p + ii - c_posb)
    out_ref[...] = rank


def _col_spec():
    return pl.BlockSpec((TIB, 1), lambda i: (i, 0))


def _row_spec():
    return pl.BlockSpec((1, NPAD), lambda i: (0, 0))


def _act_spec():
    return pl.BlockSpec((NJT, 1, TJ), lambda i: (0, 0, 0))


def _one_spec():
    return pl.BlockSpec((1, 1), lambda i: (0, 0), memory_space=pltpu.SMEM)


def kernel(nn_index1, nn_index2, x1, x2, match_score, topk):
    del topk
    pad = NPAD - N

    def padv(v, val):
        return jnp.pad(v, (0, pad), constant_values=val)

    nn1p = padv(nn_index1[0], 0)
    a1 = padv(x1[0, :, 0], 0.0)
    b1 = padv(x1[0, :, 1], 0.0)
    scp = padv(match_score[0], -3.0)

    s, a2, b2 = _sc_gather(nn1p, nn_index2[0], x2[0, :, 0], x2[0, :, 1],
                           scp)
    ja_v, sa_v, a1a_v, b1a_v, a2a_v, b2a_v, kc = _sc_compact(
        s, a1, b1, a2, b2)

    col = lambda v: v.reshape(NPAD, 1)
    act = lambda v: v.reshape(NJT, 1, TJ)

    ibs = jnp.array([p[0] for p in _TRI], jnp.int32)
    jbs = jnp.array([p[1] for p in _TRI], jnp.int32)
    grid_spec = pltpu.PrefetchScalarGridSpec(
        num_scalar_prefetch=2,
        grid=(NTRI,),
        in_specs=[pl.BlockSpec((TB, 1), lambda t, ib, jb: (ib[t], 0))] * 4
        + [pl.BlockSpec((1, TB), lambda t, ib, jb: (0, jb[t]))] * 4,
        out_specs=[
            pl.BlockSpec((1, 1), lambda t, ib, jb: (0, 0),
                         memory_space=pltpu.SMEM),
            pl.BlockSpec((1, 1), lambda t, ib, jb: (0, 0),
                         memory_space=pltpu.SMEM),
        ],
    )
    s1, s2 = pl.pallas_call(
        _sums_body,
        grid_spec=grid_spec,
        out_shape=[jax.ShapeDtypeStruct((1, 1), jnp.float32)] * 2,
    )(ibs, jbs, col(a1), col(b1), col(a2), col(b2),
      a1.reshape(1, NPAD), b1.reshape(1, NPAD),
      a2.reshape(1, NPAD), b2.reshape(1, NPAD))

    scale = 2.0 * NMS_RADIUS / (N * N)
    r1 = s1[0, 0] * scale
    r2 = s2[0, 0] * scale
    r1sq = (r1 * r1).reshape(1, 1)
    r2sq = (r2 * r2).reshape(1, 1)
    kcount = kc[0]
    nt = ((kcount + TJ - 1) // TJ).reshape(1, 1)

    grid = NPAD // TIB
    fcol = pl.pallas_call(
        _suppress_body,
        grid=(grid,),
        in_specs=[_col_spec()] * 5 + [_act_spec()] * 5
        + [_one_spec(), _one_spec(), _one_spec()],
        out_specs=_col_spec(),
        out_shape=jax.ShapeDtypeStruct((NPAD, 1), jnp.float32),
    )(col(a1), col(b1), col(a2), col(b2), col(s),
      act(a1a_v), act(b1a_v), act(a2a_v), act(b2a_v), act(sa_v),
      nt, r1sq, r2sq)

    fa_v = _sc_fgather(fcol.reshape(NPAD), ja_v, kc)

    rank = pl.pallas_call(
        _rank_body,
        grid=(grid,),
        in_specs=[_col_spec(), _act_spec(), _act_spec(), _one_spec()],
        out_specs=_col_spec(),
        out_shape=jax.ShapeDtypeStruct((NPAD, 1), jnp.int32),
    )(fcol, act(fa_v), act(ja_v), nt)

    seed1, seed2 = _sc_seeds(rank.reshape(NPAD), nn1p)

    return seed1[None, :], seed2[None, :], fcol[:N, 0][None, :]
